# baseline (speedup 1.0000x reference)
import jax
import jax.numpy as jnp
from jax.experimental import pallas as pl
from jax.experimental.pallas import tpu as pltpu

_NB = 2


def _se_fused_body(x_ref, w1_ref, w2_ref, o_ref):
    for n in range(_NB):
        x = x_ref[n]
        pooled = jnp.sum(x.astype(jnp.float32), axis=1, keepdims=True)
        s1 = jnp.maximum(
            jnp.dot(w1_ref[...], pooled, preferred_element_type=jnp.float32),
            0.0,
        )
        z = jnp.dot(w2_ref[...], s1, preferred_element_type=jnp.float32)
        gate = jax.nn.sigmoid(z).astype(x.dtype)
        o_ref[n] = x * gate


def kernel(x_nchw, w1, w2):
    N, C, H, W = x_nchw.shape
    HW = H * W
    Cr = w1.shape[0]

    w1s = w1.astype(jnp.float32) * jnp.float32(1.0 / HW)
    w2f = w2.astype(jnp.float32)

    x_flat = x_nchw.reshape(N, C, HW)

    out_flat = pl.pallas_call(
        _se_fused_body,
        out_shape=jax.ShapeDtypeStruct((N, C, HW), x_nchw.dtype),
        grid=(N // _NB,),
        in_specs=[
            pl.BlockSpec((_NB, C, HW), lambda n: (n, 0, 0)),
            pl.BlockSpec((Cr, C), lambda n: (0, 0)),
            pl.BlockSpec((C, Cr), lambda n: (0, 0)),
        ],
        out_specs=pl.BlockSpec((_NB, C, HW), lambda n: (n, 0, 0)),
        compiler_params=pltpu.CompilerParams(
            dimension_semantics=("parallel",),
            vmem_limit_bytes=64 * 1024 * 1024,
        ),
    )(x_flat, w1s, w2f)

    return out_flat.reshape(N, C, H, W)


# speedup vs baseline: 1.0037x; 1.0037x over previous
import functools

import jax
import jax.numpy as jnp
from jax import lax
from jax.experimental import pallas as pl
from jax.experimental.pallas import tpu as pltpu

_NB = 2


def _se_fused_body(x_ref, w1_ref, w2_ref, o_ref, *, inv_hw):
    x = x_ref[...]
    pooled = jnp.sum(x, axis=2) * jnp.float32(inv_hw)
    s1 = jnp.maximum(
        lax.dot_general(pooled, w1_ref[...], (((1,), (1,)), ((), ())),
                        preferred_element_type=jnp.float32),
        0.0,
    )
    z = lax.dot_general(s1, w2_ref[...], (((1,), (1,)), ((), ())),
                        preferred_element_type=jnp.float32)
    gate = jax.nn.sigmoid(z).astype(x.dtype)
    o_ref[...] = x * gate[:, :, None]


def kernel(x_nchw, w1, w2):
    N, C, H, W = x_nchw.shape
    HW = H * W
    Cr = w1.shape[0]

    x_flat = x_nchw.reshape(N, C, HW)

    out_flat = pl.pallas_call(
        functools.partial(_se_fused_body, inv_hw=1.0 / HW),
        out_shape=jax.ShapeDtypeStruct((N, C, HW), x_nchw.dtype),
        grid=(N // _NB,),
        in_specs=[
            pl.BlockSpec((_NB, C, HW), lambda n: (n, 0, 0)),
            pl.BlockSpec((Cr, C), lambda n: (0, 0)),
            pl.BlockSpec((C, Cr), lambda n: (0, 0)),
        ],
        out_specs=pl.BlockSpec((_NB, C, HW), lambda n: (n, 0, 0)),
        compiler_params=pltpu.CompilerParams(
            dimension_semantics=("parallel",),
            vmem_limit_bytes=64 * 1024 * 1024,
        ),
    )(x_flat, w1, w2)

    return out_flat.reshape(N, C, H, W)
